# Initial kernel scaffold; baseline (speedup 1.0000x reference)
import jax
import jax.numpy as jnp
from jax.experimental import pallas as pl


def kernel(feature_map, W1, b1, gamma, beta, rmean, rvar, Wcls, bcls, Woff, boff, k1, k2):
    raise NotImplementedError("write your pallas kernel here")


# speedup vs baseline: 4.4582x; 4.4582x over previous
import numpy as np
import jax
import jax.numpy as jnp
from jax import lax
from jax.experimental import pallas as pl
from jax.experimental.pallas import tpu as pltpu

F_W, F_H, F_S = 64, 48, 16
H, W, C = 48, 64, 512
HW = H * W
NA = HW * 9
TOPK = 1000
NSEL = 1024
NOUT = 300
NMS_T = 0.6
NB = 27
MBLK = 768


def _build_anchors_np():
    ws, hs = [], []
    for r in (0.5, 1.0, 2.0):
        for s in (8, 16, 32):
            size = float(F_S * s)
            ws.append(size * np.sqrt(r))
            hs.append(size / np.sqrt(r))
    ws = np.asarray(ws, dtype=np.float32)
    hs = np.asarray(hs, dtype=np.float32)
    ys, xs = np.meshgrid(np.arange(F_H), np.arange(F_W), indexing='ij')
    cx = ((xs + 0.5) * F_S).astype(np.float32)[..., None]
    cy = ((ys + 0.5) * F_S).astype(np.float32)[..., None]
    x1 = cx - ws / 2.0
    y1 = cy - hs / 2.0
    x2 = cx + ws / 2.0
    y2 = cy + hs / 2.0
    return np.stack([x1, y1, x2, y2], axis=-1).reshape(-1, 4)


_ANCHORS = _build_anchors_np()


def _conv_body(a_ref, b_ref, b1_ref, o_ref):
    acc = jnp.dot(a_ref[...], b_ref[...], preferred_element_type=jnp.float32)
    o_ref[...] = acc + b1_ref[...]


def _run_conv(A, B, b1row):
    return pl.pallas_call(
        _conv_body,
        grid=(HW // MBLK,),
        in_specs=[
            pl.BlockSpec((MBLK, 9 * C), lambda i: (i, 0)),
            pl.BlockSpec((9 * C, C), lambda i: (0, 0)),
            pl.BlockSpec((1, C), lambda i: (0, 0)),
        ],
        out_specs=pl.BlockSpec((MBLK, C), lambda i: (i, 0)),
        out_shape=jax.ShapeDtypeStruct((HW, C), jnp.float32),
    )(A, B, b1row)


def _head_body(act_ref, wh_ref, hb_ref, o_ref):
    head = jnp.dot(act_ref[...], wh_ref[...], preferred_element_type=jnp.float32)
    head = head + hb_ref[...]
    x0 = head[:, 0:16]
    x1 = head[:, 16:32]
    m = jnp.maximum(x0, x1)
    e0 = jnp.exp(x0 - m)
    e1 = jnp.exp(x1 - m)
    s = e1 / (e0 + e1)
    o_ref[...] = jnp.concatenate([s, head[:, 16:128]], axis=1)


def _run_head(actb, Whead, hbias):
    return pl.pallas_call(
        _head_body,
        in_specs=[
            pl.BlockSpec((HW, C), lambda: (0, 0)),
            pl.BlockSpec((C, 128), lambda: (0, 0)),
            pl.BlockSpec((1, 128), lambda: (0, 0)),
        ],
        out_specs=pl.BlockSpec((HW, 128), lambda: (0, 0)),
        out_shape=jax.ShapeDtypeStruct((HW, 128), jnp.float32),
    )(actb, Whead, hbias)


def _lane_excl_cumsum(x, n):
    p = jnp.pad(x, ((0, 0), (1, 0)))[:, 0:n]
    k = 1
    while k < n:
        p = p + jnp.pad(p, ((0, 0), (k, 0)))[:, 0:n]
        k *= 2
    return p


def _subl_incl_cumsum(x):
    p = x
    for k in (1, 2, 4):
        p = p + jnp.pad(p, ((k, 0), (0, 0)))[0:8]
    return p


def _blk_excl_prefix(x):
    lane = _lane_excl_cumsum(x, 128)
    rowtot = jnp.sum(x, axis=1, keepdims=True)
    prev = _subl_incl_cumsum(rowtot) - rowtot
    return lane + prev


def _hdot(a, b):
    return lax.dot_general(a, b, (((1,), (0,)), ((), ())),
                           precision=lax.Precision.HIGHEST,
                           preferred_element_type=jnp.float32)


def _topk_nms_body(s_ref, vt_ref, o_ref, gt_ref, eq_ref, sel_ref, iou_ref):
    s = s_ref[...]
    si = lax.bitcast_convert_type(s, jnp.int32)

    def bis(_, lohi):
        lo, hi = lohi
        mid = (lo + hi) // 2
        cnt = jnp.sum((si > mid).astype(jnp.float32))
        big = cnt >= float(TOPK)
        return (jnp.where(big, mid, lo), jnp.where(big, hi, mid))

    lo, hi = lax.fori_loop(0, 31, bis, (jnp.int32(-1), jnp.int32(0x7F800000)))
    V = hi
    gt = (si > V).astype(jnp.float32)
    eq = (si == V).astype(jnp.float32)
    gt_ref[...] = gt
    eq_ref[...] = eq
    need = float(TOPK) - jnp.sum(gt)
    sel_ref[...] = jnp.zeros((16, NSEL), jnp.float32)
    p_iota = lax.broadcasted_iota(jnp.int32, (1, NSEL), 1).astype(jnp.float32)

    def blk(b, carry):
        base_m, base_eq = carry
        gtb = gt_ref[pl.ds(8 * b, 8), :]
        eqb = eq_ref[pl.ds(8 * b, 8), :]
        eq_rank = base_eq + _blk_excl_prefix(eqb)
        mb = gtb + eqb * (eq_rank < need).astype(jnp.float32)
        dest = base_m + _blk_excl_prefix(mb)
        destm = jnp.where(mb > 0.0, dest, -1.0)
        destm_t = jnp.transpose(destm, (1, 0))
        vtb = vt_ref[b]
        for r in range(8):
            pt = (destm_t[:, r:r + 1] == p_iota).astype(jnp.float32)
            sel_ref[...] += _hdot(vtb[:, 128 * r:128 * (r + 1)], pt)
        return (base_m + jnp.sum(mb), base_eq + jnp.sum(eqb))

    lax.fori_loop(0, NB, blk, (jnp.float32(0.0), jnp.float32(0.0)))

    selt = sel_ref[...]
    ax1, ay1 = selt[1:2], selt[2:3]
    ax2, ay2 = selt[3:4], selt[4:5]
    dx, dy = selt[5:6], selt[6:7]
    dw, dh = selt[7:8], selt[8:9]
    valid = selt[9:10]
    aw = ax2 - ax1
    ah = ay2 - ay1
    acx = ax1 + 0.5 * aw
    acy = ay1 + 0.5 * ah
    pcx = dx * aw + acx
    pcy = dy * ah + acy
    pw = jnp.exp(dw) * aw
    ph = jnp.exp(dh) * ah
    bx1 = pcx - 0.5 * pw
    by1 = pcy - 0.5 * ph
    bx2 = pcx + 0.5 * pw
    by2 = pcy + 0.5 * ph

    s_row = selt[0:1]
    rows8 = jnp.concatenate(
        [s_row, bx1, by1, bx2, by2, valid,
         jnp.zeros((2, NSEL), jnp.float32)], axis=0)
    cols8 = jnp.transpose(rows8, (1, 0))
    s_col = cols8[:, 0:1]
    j_row = lax.broadcasted_iota(jnp.int32, (1, NSEL), 1)
    e_col = lax.broadcasted_iota(jnp.int32, (NSEL, 1), 0)
    rm = ((s_row > s_col) | ((s_row == s_col) & (j_row < e_col))).astype(jnp.float32)
    rank_col = jnp.sum(rm, axis=1, keepdims=True)
    psm = (rank_col == p_iota).astype(jnp.float32)
    srt = _hdot(rows8[1:8], psm)

    sx1, sy1 = srt[0:1], srt[1:2]
    sx2, sy2 = srt[2:3], srt[3:4]
    svalid = srt[4:5]
    scol = jnp.transpose(srt[0:7], (1, 0))
    cx1, cy1, cx2, cy2 = scol[:, 0:1], scol[:, 1:2], scol[:, 2:3], scol[:, 3:4]
    area_r = (sx2 - sx1) * (sy2 - sy1)
    area_c = (cx2 - cx1) * (cy2 - cy1)
    ix1 = jnp.maximum(cx1, sx1)
    iy1 = jnp.maximum(cy1, sy1)
    ix2 = jnp.minimum(cx2, sx2)
    iy2 = jnp.minimum(cy2, sy2)
    iw = jnp.maximum(ix2 - ix1, 0.0)
    ih = jnp.maximum(iy2 - iy1, 0.0)
    inter = iw * ih
    iou_ref[...] = inter / (area_c + area_r - inter + 1e-9)

    def nms(i, keep):
        row = iou_ref[pl.ds(i, 1), :]
        keep_i = jnp.sum(jnp.where(j_row == i, keep, 0.0))
        sup = ((row > NMS_T) & (j_row > i) & (keep_i > 0.0)).astype(jnp.float32)
        return keep * (1.0 - sup)

    keep = lax.fori_loop(0, TOPK, nms, jnp.ones((1, NSEL), jnp.float32))
    keep = keep * svalid

    dest2 = _lane_excl_cumsum(keep, NSEL)
    destk = jnp.where(keep > 0.0, dest2, -1.0)
    k8 = jnp.concatenate([destk, jnp.zeros((7, NSEL), jnp.float32)], axis=0)
    destk_col = jnp.transpose(k8, (1, 0))[:, 0:1]
    p512 = lax.broadcasted_iota(jnp.int32, (1, 512), 1).astype(jnp.float32)
    p2 = (destk_col == p512).astype(jnp.float32)
    o_ref[...] = _hdot(srt[0:4], p2)


def _run_topk_nms(s2d, vt):
    return pl.pallas_call(
        _topk_nms_body,
        out_shape=jax.ShapeDtypeStruct((4, 512), jnp.float32),
        scratch_shapes=[
            pltpu.VMEM((216, 128), jnp.float32),
            pltpu.VMEM((216, 128), jnp.float32),
            pltpu.VMEM((16, NSEL), jnp.float32),
            pltpu.VMEM((NSEL, NSEL), jnp.float32),
        ],
    )(s2d, vt)


def kernel(feature_map, W1, b1, gamma, beta, rmean, rvar,
           Wcls, bcls, Woff, boff, k1, k2):
    f32 = jnp.float32
    x = jnp.transpose(feature_map[0], (1, 2, 0))
    xp = jnp.pad(x, ((1, 1), (1, 1), (0, 0)))
    taps = [xp[dy:dy + H, dx:dx + W, :].reshape(HW, C)
            for dy in range(3) for dx in range(3)]
    A = jnp.concatenate(taps, axis=1).astype(jnp.bfloat16)
    B = jnp.concatenate(
        [jnp.transpose(W1[:, :, dy, dx]) for dy in range(3) for dx in range(3)],
        axis=0).astype(jnp.bfloat16)

    WcT = jnp.transpose(Wcls[:, :, 0, 0])
    WoT = jnp.transpose(Woff[:, :, 0, 0])
    z = jnp.zeros((C, 7), f32)
    Whead = jnp.concatenate(
        [WcT[:, 0::2], z, WcT[:, 1::2], z, WoT,
         jnp.zeros((C, 60), f32)], axis=1).astype(jnp.bfloat16)
    zb = jnp.zeros((7,), f32)
    hbias = jnp.concatenate(
        [bcls[0::2], zb, bcls[1::2], zb, boff, jnp.zeros((60,), f32)])[None, :]

    conv = _run_conv(A, B, b1[None, :])
    u = gamma[None, :] * (conv - rmean[None, :]) / jnp.sqrt(rvar[None, :] + 1e-5) \
        + beta[None, :]
    actb = jnp.maximum(u, 0.0).astype(jnp.bfloat16)
    out1 = _run_head(actb, Whead, hbias)

    scores = out1[:, 0:9].reshape(NA)
    offs = out1[:, 32:68].reshape(NA, 4)
    s2d = scores.reshape(216, 128)
    anch = jnp.asarray(_ANCHORS)
    vt = jnp.concatenate([
        scores.reshape(NB, 1, NSEL),
        jnp.transpose(anch.reshape(NB, NSEL, 4), (0, 2, 1)),
        jnp.transpose(offs.reshape(NB, NSEL, 4), (0, 2, 1)),
        jnp.ones((NB, 1, NSEL), f32),
        jnp.zeros((NB, 6, NSEL), f32),
    ], axis=1)

    res = _run_topk_nms(s2d, vt)
    return jnp.transpose(res, (1, 0))[0:NOUT, :]


# speedup vs baseline: 4.5825x; 1.0279x over previous
import numpy as np
import jax
import jax.numpy as jnp
from jax import lax
from jax.experimental import pallas as pl
from jax.experimental.pallas import tpu as pltpu

F_W, F_H, F_S = 64, 48, 16
H, W, C = 48, 64, 512
HW = H * W
NA = HW * 9
TOPK = 1000
NSEL = 1024
NOUT = 300
NMS_T = 0.6
NB = 27
MBLK = 768


def _build_anchors_np():
    ws, hs = [], []
    for r in (0.5, 1.0, 2.0):
        for s in (8, 16, 32):
            size = float(F_S * s)
            ws.append(size * np.sqrt(r))
            hs.append(size / np.sqrt(r))
    ws = np.asarray(ws, dtype=np.float32)
    hs = np.asarray(hs, dtype=np.float32)
    ys, xs = np.meshgrid(np.arange(F_H), np.arange(F_W), indexing='ij')
    cx = ((xs + 0.5) * F_S).astype(np.float32)[..., None]
    cy = ((ys + 0.5) * F_S).astype(np.float32)[..., None]
    x1 = cx - ws / 2.0
    y1 = cy - hs / 2.0
    x2 = cx + ws / 2.0
    y2 = cy + hs / 2.0
    return np.stack([x1, y1, x2, y2], axis=-1).reshape(-1, 4)


_ANCHORS = _build_anchors_np()


def _conv_body(a_ref, b_ref, b1_ref, o_ref):
    acc = jnp.dot(a_ref[...], b_ref[...], preferred_element_type=jnp.float32)
    o_ref[...] = acc + b1_ref[...]


def _run_conv(A, B, b1row):
    return pl.pallas_call(
        _conv_body,
        grid=(HW // MBLK,),
        in_specs=[
            pl.BlockSpec((MBLK, 9 * C), lambda i: (i, 0)),
            pl.BlockSpec((9 * C, C), lambda i: (0, 0)),
            pl.BlockSpec((1, C), lambda i: (0, 0)),
        ],
        out_specs=pl.BlockSpec((MBLK, C), lambda i: (i, 0)),
        out_shape=jax.ShapeDtypeStruct((HW, C), jnp.float32),
    )(A, B, b1row)


def _head_body(act_ref, wh_ref, hb_ref, o_ref):
    head = jnp.dot(act_ref[...], wh_ref[...], preferred_element_type=jnp.float32)
    head = head + hb_ref[...]
    x0 = head[:, 0:16]
    x1 = head[:, 16:32]
    m = jnp.maximum(x0, x1)
    e0 = jnp.exp(x0 - m)
    e1 = jnp.exp(x1 - m)
    s = e1 / (e0 + e1)
    o_ref[...] = jnp.concatenate([s, head[:, 16:128]], axis=1)


def _run_head(actb, Whead, hbias):
    return pl.pallas_call(
        _head_body,
        in_specs=[
            pl.BlockSpec((HW, C), lambda: (0, 0)),
            pl.BlockSpec((C, 128), lambda: (0, 0)),
            pl.BlockSpec((1, 128), lambda: (0, 0)),
        ],
        out_specs=pl.BlockSpec((HW, 128), lambda: (0, 0)),
        out_shape=jax.ShapeDtypeStruct((HW, 128), jnp.float32),
    )(actb, Whead, hbias)


def _lane_excl_cumsum(x, n):
    p = jnp.pad(x, ((0, 0), (1, 0)))[:, 0:n]
    k = 1
    while k < n:
        p = p + jnp.pad(p, ((0, 0), (k, 0)))[:, 0:n]
        k *= 2
    return p


def _subl_incl_cumsum(x):
    p = x
    for k in (1, 2, 4):
        p = p + jnp.pad(p, ((k, 0), (0, 0)))[0:8]
    return p


def _blk_excl_prefix(x):
    lane = _lane_excl_cumsum(x, 128)
    rowtot = jnp.sum(x, axis=1, keepdims=True)
    prev = _subl_incl_cumsum(rowtot) - rowtot
    return lane + prev


def _hdot(a, b):
    return lax.dot_general(a, b, (((1,), (0,)), ((), ())),
                           precision=lax.Precision.HIGHEST,
                           preferred_element_type=jnp.float32)


def _topk_nms_body(s_ref, vt_ref, o_ref, gt_ref, eq_ref, sel_ref, iou_ref):
    s = s_ref[...]
    si = lax.bitcast_convert_type(s, jnp.int32)

    def bis(_, lohi):
        lo, hi = lohi
        mid = (lo + hi) // 2
        cnt = jnp.sum((si > mid).astype(jnp.float32))
        big = cnt >= float(TOPK)
        return (jnp.where(big, mid, lo), jnp.where(big, hi, mid))

    lo, hi = lax.fori_loop(0, 31, bis, (jnp.int32(-1), jnp.int32(0x7F800000)))
    V = hi
    gt = (si > V).astype(jnp.float32)
    eq = (si == V).astype(jnp.float32)
    gt_ref[...] = gt
    eq_ref[...] = eq
    need = float(TOPK) - jnp.sum(gt)
    sel_ref[...] = jnp.zeros((16, NSEL), jnp.float32)
    p_iota = lax.broadcasted_iota(jnp.int32, (1, NSEL), 1).astype(jnp.float32)

    def blk(b, carry):
        base_m, base_eq = carry
        gtb = gt_ref[pl.ds(8 * b, 8), :]
        eqb = eq_ref[pl.ds(8 * b, 8), :]
        eq_rank = base_eq + _blk_excl_prefix(eqb)
        mb = gtb + eqb * (eq_rank < need).astype(jnp.float32)
        dest = base_m + _blk_excl_prefix(mb)
        destm = jnp.where(mb > 0.0, dest, -1.0)
        destm_t = jnp.transpose(destm, (1, 0))
        vtb = vt_ref[b]
        pt = jnp.concatenate(
            [(destm_t[:, r:r + 1] == p_iota).astype(jnp.float32)
             for r in range(8)], axis=0)
        sel_ref[...] += _hdot(vtb, pt)
        return (base_m + jnp.sum(mb), base_eq + jnp.sum(eqb))

    lax.fori_loop(0, NB, blk, (jnp.float32(0.0), jnp.float32(0.0)))

    selt = sel_ref[...]
    ax1, ay1 = selt[1:2], selt[2:3]
    ax2, ay2 = selt[3:4], selt[4:5]
    dx, dy = selt[5:6], selt[6:7]
    dw, dh = selt[7:8], selt[8:9]
    valid = selt[9:10]
    aw = ax2 - ax1
    ah = ay2 - ay1
    acx = ax1 + 0.5 * aw
    acy = ay1 + 0.5 * ah
    pcx = dx * aw + acx
    pcy = dy * ah + acy
    pw = jnp.exp(dw) * aw
    ph = jnp.exp(dh) * ah
    bx1 = pcx - 0.5 * pw
    by1 = pcy - 0.5 * ph
    bx2 = pcx + 0.5 * pw
    by2 = pcy + 0.5 * ph

    s_row = selt[0:1]
    rows8 = jnp.concatenate(
        [s_row, bx1, by1, bx2, by2, valid,
         jnp.zeros((2, NSEL), jnp.float32)], axis=0)
    cols8 = jnp.transpose(rows8, (1, 0))
    s_col = cols8[:, 0:1]
    j_row = lax.broadcasted_iota(jnp.int32, (1, NSEL), 1)
    e_col = lax.broadcasted_iota(jnp.int32, (NSEL, 1), 0)
    rm = ((s_row > s_col) | ((s_row == s_col) & (j_row < e_col))).astype(jnp.float32)
    rank_col = jnp.sum(rm, axis=1, keepdims=True)
    psm = (rank_col == p_iota).astype(jnp.float32)
    srt = _hdot(rows8[1:8], psm)

    sx1, sy1 = srt[0:1], srt[1:2]
    sx2, sy2 = srt[2:3], srt[3:4]
    svalid = srt[4:5]
    scol = jnp.transpose(srt[0:7], (1, 0))
    cx1, cy1, cx2, cy2 = scol[:, 0:1], scol[:, 1:2], scol[:, 2:3], scol[:, 3:4]
    area_r = (sx2 - sx1) * (sy2 - sy1)
    area_c = (cx2 - cx1) * (cy2 - cy1)
    ix1 = jnp.maximum(cx1, sx1)
    iy1 = jnp.maximum(cy1, sy1)
    ix2 = jnp.minimum(cx2, sx2)
    iy2 = jnp.minimum(cy2, sy2)
    iw = jnp.maximum(ix2 - ix1, 0.0)
    ih = jnp.maximum(iy2 - iy1, 0.0)
    inter = iw * ih
    iou = inter / (area_c + area_r - inter + 1e-9)
    iou_ref[...] = iou.reshape(NSEL, 8, 128)

    idx8 = (lax.broadcasted_iota(jnp.int32, (8, 128), 0) * 128
            + lax.broadcasted_iota(jnp.int32, (8, 128), 1))

    def nms(i, keep):
        row8 = iou_ref[i]
        keep_i = jnp.sum(jnp.where(idx8 == i, keep, 0.0))
        sup = ((row8 > NMS_T) & (idx8 > i) & (keep_i > 0.0)).astype(jnp.float32)
        return keep * (1.0 - sup)

    keep = lax.fori_loop(0, TOPK, nms, jnp.ones((8, 128), jnp.float32))
    keep = keep * (idx8 < TOPK).astype(jnp.float32)

    dest2 = _blk_excl_prefix(keep)
    destk = jnp.where(keep > 0.0, dest2, -1.0)
    destk_t = jnp.transpose(destk, (1, 0))
    p512 = lax.broadcasted_iota(jnp.int32, (1, 512), 1).astype(jnp.float32)
    p2 = jnp.concatenate(
        [(destk_t[:, r:r + 1] == p512).astype(jnp.float32)
         for r in range(8)], axis=0)
    o_ref[...] = _hdot(srt[0:4], p2)


def _run_topk_nms(s2d, vt):
    return pl.pallas_call(
        _topk_nms_body,
        out_shape=jax.ShapeDtypeStruct((4, 512), jnp.float32),
        scratch_shapes=[
            pltpu.VMEM((216, 128), jnp.float32),
            pltpu.VMEM((216, 128), jnp.float32),
            pltpu.VMEM((16, NSEL), jnp.float32),
            pltpu.VMEM((NSEL, 8, 128), jnp.float32),
        ],
    )(s2d, vt)


def kernel(feature_map, W1, b1, gamma, beta, rmean, rvar,
           Wcls, bcls, Woff, boff, k1, k2):
    f32 = jnp.float32
    x = jnp.transpose(feature_map[0], (1, 2, 0))
    xp = jnp.pad(x, ((1, 1), (1, 1), (0, 0)))
    taps = [xp[dy:dy + H, dx:dx + W, :].reshape(HW, C)
            for dy in range(3) for dx in range(3)]
    A = jnp.concatenate(taps, axis=1).astype(jnp.bfloat16)
    B = jnp.concatenate(
        [jnp.transpose(W1[:, :, dy, dx]) for dy in range(3) for dx in range(3)],
        axis=0).astype(jnp.bfloat16)

    WcT = jnp.transpose(Wcls[:, :, 0, 0])
    WoT = jnp.transpose(Woff[:, :, 0, 0])
    z = jnp.zeros((C, 7), f32)
    Whead = jnp.concatenate(
        [WcT[:, 0::2], z, WcT[:, 1::2], z, WoT,
         jnp.zeros((C, 60), f32)], axis=1).astype(jnp.bfloat16)
    zb = jnp.zeros((7,), f32)
    hbias = jnp.concatenate(
        [bcls[0::2], zb, bcls[1::2], zb, boff, jnp.zeros((60,), f32)])[None, :]

    conv = _run_conv(A, B, b1[None, :])
    u = gamma[None, :] * (conv - rmean[None, :]) / jnp.sqrt(rvar[None, :] + 1e-5) \
        + beta[None, :]
    actb = jnp.maximum(u, 0.0).astype(jnp.bfloat16)
    out1 = _run_head(actb, Whead, hbias)

    scores = out1[:, 0:9].reshape(NA)
    offs = out1[:, 32:68].reshape(NA, 4)
    s2d = scores.reshape(216, 128)
    anch = jnp.asarray(_ANCHORS)
    vt = jnp.concatenate([
        scores.reshape(NB, 1, NSEL),
        jnp.transpose(anch.reshape(NB, NSEL, 4), (0, 2, 1)),
        jnp.transpose(offs.reshape(NB, NSEL, 4), (0, 2, 1)),
        jnp.ones((NB, 1, NSEL), f32),
        jnp.zeros((NB, 6, NSEL), f32),
    ], axis=1)

    res = _run_topk_nms(s2d, vt)
    return jnp.transpose(res, (1, 0))[0:NOUT, :]


# speedup vs baseline: 5.1845x; 1.1314x over previous
import numpy as np
import jax
import jax.numpy as jnp
from jax import lax
from jax.experimental import pallas as pl
from jax.experimental.pallas import tpu as pltpu

F_W, F_H, F_S = 64, 48, 16
H, W, C = 48, 64, 512
HW = H * W
NA = HW * 9
TOPK = 1000
NSEL = 1024
NOUT = 300
NMS_T = 0.6
NB = 27
MBLK = 768


def _build_anchors_np():
    ws, hs = [], []
    for r in (0.5, 1.0, 2.0):
        for s in (8, 16, 32):
            size = float(F_S * s)
            ws.append(size * np.sqrt(r))
            hs.append(size / np.sqrt(r))
    ws = np.asarray(ws, dtype=np.float32)
    hs = np.asarray(hs, dtype=np.float32)
    ys, xs = np.meshgrid(np.arange(F_H), np.arange(F_W), indexing='ij')
    cx = ((xs + 0.5) * F_S).astype(np.float32)[..., None]
    cy = ((ys + 0.5) * F_S).astype(np.float32)[..., None]
    x1 = cx - ws / 2.0
    y1 = cy - hs / 2.0
    x2 = cx + ws / 2.0
    y2 = cy + hs / 2.0
    return np.stack([x1, y1, x2, y2], axis=-1).reshape(-1, 4)


_ANCHORS = _build_anchors_np()


def _conv_body(a_ref, b_ref, b1_ref, o_ref):
    acc = jnp.dot(a_ref[...], b_ref[...], preferred_element_type=jnp.float32)
    o_ref[...] = acc + b1_ref[...]


def _run_conv(A, B, b1row):
    return pl.pallas_call(
        _conv_body,
        grid=(HW // MBLK,),
        in_specs=[
            pl.BlockSpec((MBLK, 9 * C), lambda i: (i, 0)),
            pl.BlockSpec((9 * C, C), lambda i: (0, 0)),
            pl.BlockSpec((1, C), lambda i: (0, 0)),
        ],
        out_specs=pl.BlockSpec((MBLK, C), lambda i: (i, 0)),
        out_shape=jax.ShapeDtypeStruct((HW, C), jnp.float32),
    )(A, B, b1row)


def _head_body(act_ref, wh_ref, hb_ref, o_ref):
    head = jnp.dot(act_ref[...], wh_ref[...], preferred_element_type=jnp.float32)
    head = head + hb_ref[...]
    x0 = head[:, 0:16]
    x1 = head[:, 16:32]
    m = jnp.maximum(x0, x1)
    e0 = jnp.exp(x0 - m)
    e1 = jnp.exp(x1 - m)
    s = e1 / (e0 + e1)
    o_ref[...] = jnp.concatenate([s, head[:, 16:128]], axis=1)


def _run_head(actb, Whead, hbias):
    return pl.pallas_call(
        _head_body,
        in_specs=[
            pl.BlockSpec((HW, C), lambda: (0, 0)),
            pl.BlockSpec((C, 128), lambda: (0, 0)),
            pl.BlockSpec((1, 128), lambda: (0, 0)),
        ],
        out_specs=pl.BlockSpec((HW, 128), lambda: (0, 0)),
        out_shape=jax.ShapeDtypeStruct((HW, 128), jnp.float32),
    )(actb, Whead, hbias)


def _lane_excl_cumsum(x, n):
    p = jnp.pad(x, ((0, 0), (1, 0)))[:, 0:n]
    k = 1
    while k < n:
        p = p + jnp.pad(p, ((0, 0), (k, 0)))[:, 0:n]
        k *= 2
    return p


def _subl_incl_cumsum(x):
    p = x
    for k in (1, 2, 4):
        p = p + jnp.pad(p, ((k, 0), (0, 0)))[0:8]
    return p


def _blk_excl_prefix(x):
    lane = _lane_excl_cumsum(x, 128)
    rowtot = jnp.sum(x, axis=1, keepdims=True)
    prev = _subl_incl_cumsum(rowtot) - rowtot
    return lane + prev


def _hdot(a, b):
    return lax.dot_general(a, b, (((1,), (0,)), ((), ())),
                           precision=lax.Precision.HIGHEST,
                           preferred_element_type=jnp.float32)


def _topk_nms_body(s_ref, vt_ref, o_ref, gt_ref, eq_ref, sel_ref, iou_ref):
    s = s_ref[...]
    si = lax.bitcast_convert_type(s, jnp.int32)

    def bis(_, lohi):
        lo, hi = lohi
        mid = (lo + hi) // 2
        cnt = jnp.sum((si > mid).astype(jnp.float32))
        big = cnt >= float(TOPK)
        return (jnp.where(big, mid, lo), jnp.where(big, hi, mid))

    lo, hi = lax.fori_loop(0, 31, bis, (jnp.int32(-1), jnp.int32(0x7F800000)))
    V = hi
    gt = (si > V).astype(jnp.float32)
    eq = (si == V).astype(jnp.float32)
    gt_ref[...] = gt
    eq_ref[...] = eq
    need = float(TOPK) - jnp.sum(gt)
    sel_ref[...] = jnp.zeros((48, NSEL), jnp.float32)
    p_iota = lax.broadcasted_iota(jnp.int32, (1, NSEL), 1).astype(jnp.float32)

    def blk(b, carry):
        base_m, base_eq = carry
        gtb = gt_ref[pl.ds(8 * b, 8), :]
        eqb = eq_ref[pl.ds(8 * b, 8), :]
        eq_rank = base_eq + _blk_excl_prefix(eqb)
        mb = gtb + eqb * (eq_rank < need).astype(jnp.float32)
        dest = base_m + _blk_excl_prefix(mb)
        destm = jnp.where(mb > 0.0, dest, -1.0)
        destm_t = jnp.transpose(destm, (1, 0))
        vtb = vt_ref[b]
        pt = jnp.concatenate(
            [(destm_t[:, r:r + 1] == p_iota).astype(jnp.bfloat16)
             for r in range(8)], axis=0)
        sel_ref[...] += jnp.dot(vtb, pt, preferred_element_type=jnp.float32)
        return (base_m + jnp.sum(mb), base_eq + jnp.sum(eqb))

    lax.fori_loop(0, NB, blk, (jnp.float32(0.0), jnp.float32(0.0)))

    s3 = sel_ref[...]
    selt = s3[0:16] + s3[16:32] + s3[32:48]
    ax1, ay1 = selt[1:2], selt[2:3]
    ax2, ay2 = selt[3:4], selt[4:5]
    dx, dy = selt[5:6], selt[6:7]
    dw, dh = selt[7:8], selt[8:9]
    valid = selt[9:10]
    aw = ax2 - ax1
    ah = ay2 - ay1
    acx = ax1 + 0.5 * aw
    acy = ay1 + 0.5 * ah
    pcx = dx * aw + acx
    pcy = dy * ah + acy
    pw = jnp.exp(dw) * aw
    ph = jnp.exp(dh) * ah
    bx1 = pcx - 0.5 * pw
    by1 = pcy - 0.5 * ph
    bx2 = pcx + 0.5 * pw
    by2 = pcy + 0.5 * ph

    s_row = selt[0:1]
    rows8 = jnp.concatenate(
        [s_row, bx1, by1, bx2, by2, valid,
         jnp.zeros((2, NSEL), jnp.float32)], axis=0)
    cols8 = jnp.transpose(rows8, (1, 0))
    s_col = cols8[:, 0:1]
    j_row = lax.broadcasted_iota(jnp.int32, (1, NSEL), 1)
    e_col = lax.broadcasted_iota(jnp.int32, (NSEL, 1), 0)
    rm = ((s_row > s_col) | ((s_row == s_col) & (j_row < e_col))).astype(jnp.float32)
    rank_col = jnp.sum(rm, axis=1, keepdims=True)
    psm = (rank_col == p_iota).astype(jnp.float32)
    srt = _hdot(rows8[1:8], psm)

    sx1, sy1 = srt[0:1], srt[1:2]
    sx2, sy2 = srt[2:3], srt[3:4]
    svalid = srt[4:5]
    scol = jnp.transpose(srt[0:7], (1, 0))
    cx1, cy1, cx2, cy2 = scol[:, 0:1], scol[:, 1:2], scol[:, 2:3], scol[:, 3:4]
    area_r = (sx2 - sx1) * (sy2 - sy1)
    area_c = (cx2 - cx1) * (cy2 - cy1)
    ix1 = jnp.maximum(cx1, sx1)
    iy1 = jnp.maximum(cy1, sy1)
    ix2 = jnp.minimum(cx2, sx2)
    iy2 = jnp.minimum(cy2, sy2)
    iw = jnp.maximum(ix2 - ix1, 0.0)
    ih = jnp.maximum(iy2 - iy1, 0.0)
    inter = iw * ih
    iou = inter / (area_c + area_r - inter + 1e-9)
    iou_ref[...] = iou.reshape(NSEL, 8, 128)

    idx8 = (lax.broadcasted_iota(jnp.int32, (8, 128), 0) * 128
            + lax.broadcasted_iota(jnp.int32, (8, 128), 1))

    def nms(i, keep):
        row8 = iou_ref[i]
        keep_i = jnp.sum(jnp.where(idx8 == i, keep, 0.0))
        sup = ((row8 > NMS_T) & (idx8 > i) & (keep_i > 0.0)).astype(jnp.float32)
        return keep * (1.0 - sup)

    keep = lax.fori_loop(0, TOPK, nms, jnp.ones((8, 128), jnp.float32))
    keep = keep * (idx8 < TOPK).astype(jnp.float32)

    dest2 = _blk_excl_prefix(keep)
    destk = jnp.where(keep > 0.0, dest2, -1.0)
    destk_t = jnp.transpose(destk, (1, 0))
    p512 = lax.broadcasted_iota(jnp.int32, (1, 512), 1).astype(jnp.float32)
    p2 = jnp.concatenate(
        [(destk_t[:, r:r + 1] == p512).astype(jnp.float32)
         for r in range(8)], axis=0)
    o_ref[...] = _hdot(srt[0:4], p2)


def _run_topk_nms(s2d, vt):
    return pl.pallas_call(
        _topk_nms_body,
        out_shape=jax.ShapeDtypeStruct((4, 512), jnp.float32),
        scratch_shapes=[
            pltpu.VMEM((216, 128), jnp.float32),
            pltpu.VMEM((216, 128), jnp.float32),
            pltpu.VMEM((48, NSEL), jnp.float32),
            pltpu.VMEM((NSEL, 8, 128), jnp.float32),
        ],
    )(s2d, vt)


def kernel(feature_map, W1, b1, gamma, beta, rmean, rvar,
           Wcls, bcls, Woff, boff, k1, k2):
    f32 = jnp.float32
    x = jnp.transpose(feature_map[0], (1, 2, 0))
    xp = jnp.pad(x, ((1, 1), (1, 1), (0, 0))).astype(jnp.bfloat16)
    taps = [xp[dy:dy + H, dx:dx + W, :].reshape(HW, C)
            for dy in range(3) for dx in range(3)]
    A = jnp.concatenate(taps, axis=1)
    B = jnp.concatenate(
        [jnp.transpose(W1[:, :, dy, dx]) for dy in range(3) for dx in range(3)],
        axis=0).astype(jnp.bfloat16)

    WcT = jnp.transpose(Wcls[:, :, 0, 0])
    WoT = jnp.transpose(Woff[:, :, 0, 0])
    z = jnp.zeros((C, 7), f32)
    Whead = jnp.concatenate(
        [WcT[:, 0::2], z, WcT[:, 1::2], z, WoT,
         jnp.zeros((C, 60), f32)], axis=1).astype(jnp.bfloat16)
    zb = jnp.zeros((7,), f32)
    hbias = jnp.concatenate(
        [bcls[0::2], zb, bcls[1::2], zb, boff, jnp.zeros((60,), f32)])[None, :]

    conv = _run_conv(A, B, b1[None, :])
    u = gamma[None, :] * (conv - rmean[None, :]) / jnp.sqrt(rvar[None, :] + 1e-5) \
        + beta[None, :]
    actb = jnp.maximum(u, 0.0).astype(jnp.bfloat16)
    out1 = _run_head(actb, Whead, hbias)

    scores = out1[:, 0:9].reshape(NA)
    offs = out1[:, 32:68].reshape(NA, 4)
    s2d = scores.reshape(216, 128)
    anch = jnp.asarray(_ANCHORS)
    vt = jnp.concatenate([
        scores.reshape(NB, 1, NSEL),
        jnp.transpose(anch.reshape(NB, NSEL, 4), (0, 2, 1)),
        jnp.transpose(offs.reshape(NB, NSEL, 4), (0, 2, 1)),
        jnp.ones((NB, 1, NSEL), f32),
        jnp.zeros((NB, 6, NSEL), f32),
    ], axis=1)

    vh = vt.astype(jnp.bfloat16)
    rm1 = vt - vh.astype(f32)
    vm = rm1.astype(jnp.bfloat16)
    vl = (rm1 - vm.astype(f32)).astype(jnp.bfloat16)
    vt3 = jnp.concatenate([vh, vm, vl], axis=1)

    res = _run_topk_nms(s2d, vt3)
    return jnp.transpose(res, (1, 0))[0:NOUT, :]
